# baseline (speedup 1.0000x reference)
import jax
import jax.numpy as jnp
from jax.experimental import pallas as pl
from jax.experimental.pallas import tpu as pltpu

_SEQ = 4096
_TST = 2048
_RL = 256
_NC = _SEQ // _RL
_OS = 128
_SL = 8
_NSLAB = _OS // _SL
_NG = _SEQ // _OS
_NT = _TST // _OS + 2
_ACCR = (_NG + 2) * _OS

_S2PI_HI = 6.28125
_S2PI_LO = 0.0019353071795864846
_SINV2PI = 0.15915494309189535
_SIN_C = (0.9999999528369572, -0.16666629704656394, 0.008332868373268382,
          -0.00019819995093551526, 2.7117597258194404e-06,
          -2.0823799434799284e-08)
_COS_C = (0.9999994009689195, -0.4999953021394909, 0.04166075139470328,
          -0.0013861784143072344, 2.4240032927225208e-05,
          -2.2132124788409868e-07)


def _reduce_2pi(theta):
    n = jnp.floor(theta * _SINV2PI + 0.5)
    return (theta - n * _S2PI_HI) - n * _S2PI_LO


def _poly_even(r2, coeffs):
    p = coeffs[-1]
    for c in coeffs[-2::-1]:
        p = c + r2 * p
    return p


def _fast_sincos(theta):
    r = _reduce_2pi(theta)
    r2 = r * r
    return r * _poly_even(r2, _SIN_C), _poly_even(r2, _COS_C)


def _imu_body(k_ref, d_ref, phi_ref, c_ref, kt_ref, dt_ref, phit_ref, ct_ref,
              out_ref, a_scr, w_scr, at_scr, wt_scr,
              s8_scr, c8_scr, s8t_scr, c8t_scr,
              ag_scr, bg_scr, agt_scr, bgt_scr, acc_scr):
    a_scr[...] = d_ref[...] * -0.5
    w_scr[...] = jnp.sqrt(k_ref[...] * 4.0 - d_ref[...] * d_ref[...]) * 0.5
    at_scr[...] = dt_ref[...] * -0.5
    wt_scr[...] = jnp.sqrt(kt_ref[...] * 4.0 - dt_ref[...] * dt_ref[...]) * 0.5
    s8, c8 = _fast_sincos(w_scr[...] * float(_SL))
    e8 = jnp.exp(a_scr[...] * float(_SL))
    s8_scr[...] = s8 * e8
    c8_scr[...] = c8 * e8
    s8t, c8t = _fast_sincos(wt_scr[...] * float(_SL))
    e8t = jnp.exp(at_scr[...] * float(_SL))
    s8t_scr[...] = s8t * e8t
    c8t_scr[...] = c8t * e8t
    sg, cg = _fast_sincos(w_scr[...] * float(_OS))
    eg = jnp.exp(a_scr[...] * float(_OS))
    ag_scr[...] = (cg + cg) * eg
    bg_scr[...] = eg * eg
    sgt, cgt = _fast_sincos(wt_scr[...] * float(_OS))
    egt = jnp.exp(at_scr[...] * float(_OS))
    agt_scr[...] = (cgt + cgt) * egt
    bgt_scr[...] = egt * egt

    acc_scr[...] = jnp.zeros((_ACCR, _RL), jnp.float32)

    sub = jax.lax.broadcasted_iota(jnp.int32, (_SL, _RL), 0).astype(jnp.float32)
    lane = jax.lax.broadcasted_iota(jnp.int32, (_SL, _RL), 1).astype(jnp.float32)
    sml = sub - lane

    def chunk_body(c, carry):
        a = a_scr[pl.ds(c, 1), :]
        w = w_scr[pl.ds(c, 1), :]
        ph = phi_ref[pl.ds(c, 1), :]
        cc = c_ref[pl.ds(c, 1), :]
        at = at_scr[pl.ds(c, 1), :]
        wt = wt_scr[pl.ds(c, 1), :]
        pht = phit_ref[pl.ds(c, 1), :]
        ct = ct_ref[pl.ds(c, 1), :]
        s8e = s8_scr[pl.ds(c, 1), :]
        c8e = c8_scr[pl.ds(c, 1), :]
        s8te = s8t_scr[pl.ds(c, 1), :]
        c8te = c8t_scr[pl.ds(c, 1), :]
        ag = ag_scr[pl.ds(c, 1), :]
        bg = bg_scr[pl.ds(c, 1), :]
        agt = agt_scr[pl.ds(c, 1), :]
        bgt = bgt_scr[pl.ds(c, 1), :]
        a8 = c8e + c8e
        b8 = c8e * c8e + s8e * s8e
        a8t = c8te + c8te
        b8t = c8te * c8te + s8te * s8te
        jb = 2 * c

        def eval_tile(j):
            t0 = (j * _OS - c * _RL).astype(jnp.float32) + sml
            s, co = _fast_sincos(t0 * w + ph)
            e = cc * jnp.exp(a * t0)
            st, cot = _fast_sincos(t0 * wt + pht)
            et = ct * jnp.exp(at * t0)
            x0 = e * s
            x1 = x0 * c8e + (e * co) * s8e
            y0 = et * st
            y1 = y0 * c8te + (et * cot) * s8te
            return t0, (x0, x1, y0, y1)

        def accum_tile(j, t0, lines, masked):
            cx, nx, cy, ny = lines
            for v in range(_NSLAB):
                val = cx + cy
                if masked:
                    tv = t0 + float(v * _SL)
                    valid = (tv >= 0.0) & (tv < float(_TST))
                    val = jnp.where(valid, val, 0.0)
                acc_scr[pl.ds(j * _OS + v * _SL, _SL), :] += val
                if v + 1 < _NSLAB:
                    cx, nx = nx, a8 * nx - b8 * cx
                    cy, ny = ny, a8t * ny - b8t * cy
            return None

        t0h, lines_h = eval_tile(jb)
        accum_tile(jb, t0h, lines_h, True)
        t0h, lines_h = eval_tile(jb + 1)
        accum_tile(jb + 1, t0h, lines_h, True)
        _, prev = eval_tile(jb + 2)
        accum_tile(jb + 2, None, prev, False)
        _, cur = eval_tile(jb + 3)
        accum_tile(jb + 3, None, cur, False)

        def march(j, lines):
            px0, px1, py0, py1, cx0, cx1, cy0, cy1 = lines
            nx0 = ag * cx0 - bg * px0
            nx1 = ag * cx1 - bg * px1
            ny0 = agt * cy0 - bgt * py0
            ny1 = agt * cy1 - bgt * py1
            return (cx0, cx1, cy0, cy1, nx0, nx1, ny0, ny1)

        def march_clean(j, lines):
            lines = march(j, lines)
            accum_tile(j, None, lines[4:], False)
            return lines

        def march_masked(j, lines):
            lines = march(j, lines)
            t0 = (j * _OS - c * _RL).astype(jnp.float32) + sml
            accum_tile(j, t0, lines[4:], True)
            return lines

        lines = prev + cur
        jm = jnp.minimum(jb + _NT - 2, _NG)
        jt = jnp.minimum(jb + _NT, _NG)
        lines = jax.lax.fori_loop(jb + 4, jm, march_clean, lines)
        jax.lax.fori_loop(jm, jt, march_masked, lines)
        return carry

    jax.lax.fori_loop(0, _NC, chunk_body, 0)

    def reduce_body(r, carry):
        out_ref[pl.ds(r, 1), :] = (
            jnp.sum(acc_scr[pl.ds(r * _SL, _SL), :], axis=1).reshape(1, _SL))
        return carry

    jax.lax.fori_loop(0, _SEQ // _SL, reduce_body, 0)


def kernel(k_imu, d_imu, phi_imu, c_imu, k_theta_imu, d_theta_imu,
           phi_theta_imu, c_theta_imu, seq_len,
           time_steps_propogate_kinematics):
    shape2 = (_NC, _RL)
    args = [jnp.asarray(x, jnp.float32).reshape(shape2) for x in
            (k_imu, d_imu, phi_imu, c_imu, k_theta_imu, d_theta_imu,
             phi_theta_imu, c_theta_imu)]
    out = pl.pallas_call(
        _imu_body,
        out_shape=jax.ShapeDtypeStruct((_SEQ // _SL, _SL), jnp.float32),
        scratch_shapes=[pltpu.VMEM((_NC, _RL), jnp.float32)] * 12
        + [pltpu.VMEM((_ACCR, _RL), jnp.float32)],
    )(*args)
    return out.reshape(1, _SEQ)


# speedup vs baseline: 2.0342x; 2.0342x over previous
import jax
import jax.numpy as jnp
from jax.experimental import pallas as pl
from jax.experimental.pallas import tpu as pltpu

_SEQ = 4096
_TST = 2048
_RL = 256
_NC = _SEQ // _RL
_OS = 128
_SL = 8
_NSLAB = _OS // _SL
_NG = _SEQ // _OS

_S2PI_HI = 6.28125
_S2PI_LO = 0.0019353071795864846
_SINV2PI = 0.15915494309189535
_SIN_C = (0.9999999528369572, -0.16666629704656394, 0.008332868373268382,
          -0.00019819995093551526, 2.7117597258194404e-06,
          -2.0823799434799284e-08)
_COS_C = (0.9999994009689195, -0.4999953021394909, 0.04166075139470328,
          -0.0013861784143072344, 2.4240032927225208e-05,
          -2.2132124788409868e-07)


def _reduce_2pi(theta):
    n = jnp.floor(theta * _SINV2PI + 0.5)
    return (theta - n * _S2PI_HI) - n * _S2PI_LO


def _poly_even(r2, coeffs):
    p = coeffs[-1]
    for c in coeffs[-2::-1]:
        p = c + r2 * p
    return p


def _fast_sincos(theta):
    r = _reduce_2pi(theta)
    r2 = r * r
    return r * _poly_even(r2, _SIN_C), _poly_even(r2, _COS_C)


def _imu_body(k_ref, d_ref, phi_ref, c_ref, kt_ref, dt_ref, phit_ref, ct_ref,
              out_ref, a_scr, w_scr, at_scr, wt_scr,
              s8_scr, c8_scr, s8t_scr, c8t_scr, acc_scr):
    a_scr[...] = d_ref[...] * -0.5
    w_scr[...] = jnp.sqrt(k_ref[...] * 4.0 - d_ref[...] * d_ref[...]) * 0.5
    at_scr[...] = dt_ref[...] * -0.5
    wt_scr[...] = jnp.sqrt(kt_ref[...] * 4.0 - dt_ref[...] * dt_ref[...]) * 0.5
    s8, c8 = _fast_sincos(w_scr[...] * float(_SL))
    e8 = jnp.exp(a_scr[...] * float(_SL))
    s8_scr[...] = s8 * e8
    c8_scr[...] = c8 * e8
    s8t, c8t = _fast_sincos(wt_scr[...] * float(_SL))
    e8t = jnp.exp(at_scr[...] * float(_SL))
    s8t_scr[...] = s8t * e8t
    c8t_scr[...] = c8t * e8t

    sub = jax.lax.broadcasted_iota(jnp.int32, (_SL, _RL), 0).astype(jnp.float32)
    lane = jax.lax.broadcasted_iota(jnp.int32, (_SL, _RL), 1).astype(jnp.float32)
    sml = sub - lane

    def group_body(j, carry):
        p0 = j * _OS
        c_lo = jnp.maximum(p0 - (_TST - 1), 0) // _RL
        c_hi = (p0 + _OS - 1) // _RL
        i_lo = jnp.clip((p0 + _OS - _TST + _RL - 1) // _RL, c_lo, c_hi + 1)
        i_hi = jnp.clip((p0 - (_RL - 1)) // _RL + 1, i_lo, c_hi + 1)

        def init_chunk(c):
            base = (p0 - c * _RL).astype(jnp.float32)
            t0 = base + sml
            a = a_scr[pl.ds(c, 1), :]
            w = w_scr[pl.ds(c, 1), :]
            ph = phi_ref[pl.ds(c, 1), :]
            cc = c_ref[pl.ds(c, 1), :]
            at = at_scr[pl.ds(c, 1), :]
            wt = wt_scr[pl.ds(c, 1), :]
            pht = phit_ref[pl.ds(c, 1), :]
            ct = ct_ref[pl.ds(c, 1), :]
            s8 = s8_scr[pl.ds(c, 1), :]
            c8 = c8_scr[pl.ds(c, 1), :]
            s8t = s8t_scr[pl.ds(c, 1), :]
            c8t = c8t_scr[pl.ds(c, 1), :]
            s, co = _fast_sincos(t0 * w + ph)
            e = cc * jnp.exp(a * t0)
            st, cot = _fast_sincos(t0 * wt + pht)
            et = ct * jnp.exp(at * t0)
            x0 = e * s
            x1 = x0 * c8 + (e * co) * s8
            y0 = et * st
            y1 = y0 * c8t + (et * cot) * s8t
            coef = (c8 + c8, c8 * c8 + s8 * s8,
                    c8t + c8t, c8t * c8t + s8t * s8t)
            return t0, [x0, x1, y0, y1], coef

        def step(state, coef):
            x0, x1, y0, y1 = state
            al, bl, at_, bt = coef
            return [x1, al * x1 - bl * x0, y1, at_ * y1 - bt * y0]

        def chunk_masked(c, _):
            t0, st, coef = init_chunk(c)
            for v in range(_NSLAB):
                tv = t0 + float(v * _SL)
                valid = (tv >= 0.0) & (tv < float(_TST))
                sl = slice(v * _SL, (v + 1) * _SL)
                acc_scr[sl, :] += jnp.where(valid, st[0] + st[2], 0.0)
                if v + 1 < _NSLAB:
                    st = step(st, coef)
            return 0

        def chunk_clean(c, _):
            t0, st, coef = init_chunk(c)
            for v in range(_NSLAB):
                sl = slice(v * _SL, (v + 1) * _SL)
                acc_scr[sl, :] += st[0] + st[2]
                if v + 1 < _NSLAB:
                    st = step(st, coef)
            return 0

        acc_scr[...] = jnp.zeros((_OS, _RL), jnp.float32)
        jax.lax.fori_loop(c_lo, i_lo, chunk_masked, 0)
        jax.lax.fori_loop(i_lo, i_hi, chunk_clean, 0)
        jax.lax.fori_loop(i_hi, c_hi + 1, chunk_masked, 0)
        for v in range(_NSLAB):
            out_ref[pl.ds(j * _NSLAB + v, 1), :] = (
                jnp.sum(acc_scr[v * _SL:(v + 1) * _SL, :], axis=1)
                .reshape(1, _SL))
        return carry

    jax.lax.fori_loop(0, _NG, group_body, 0)


def kernel(k_imu, d_imu, phi_imu, c_imu, k_theta_imu, d_theta_imu,
           phi_theta_imu, c_theta_imu, seq_len,
           time_steps_propogate_kinematics):
    shape2 = (_NC, _RL)
    args = [jnp.asarray(x, jnp.float32).reshape(shape2) for x in
            (k_imu, d_imu, phi_imu, c_imu, k_theta_imu, d_theta_imu,
             phi_theta_imu, c_theta_imu)]
    out = pl.pallas_call(
        _imu_body,
        out_shape=jax.ShapeDtypeStruct((_SEQ // _SL, _SL), jnp.float32),
        scratch_shapes=[pltpu.VMEM((_NC, _RL), jnp.float32)] * 8
        + [pltpu.VMEM((_OS, _RL), jnp.float32)],
    )(*args)
    return out.reshape(1, _SEQ)
